# baseline (speedup 1.0000x reference)
import jax
import jax.numpy as jnp
from jax.experimental import pallas as pl
from jax.experimental.pallas import tpu as pltpu

_W = 16


def _dma_body(perm_ref, in_hbm, out_hbm, sems):
    C = in_hbm.shape[0]

    def mk(i):
        return pltpu.make_async_copy(
            in_hbm.at[perm_ref[i]], out_hbm.at[i], sems.at[i % _W]
        )

    for i in range(_W):
        mk(i).start()

    def lbody(i, _):
        mk(i - _W).wait()
        mk(i).start()
        return 0

    jax.lax.fori_loop(_W, C, lbody, 0)
    for j in range(C - _W, C):
        mk(j).wait()


def kernel(inp):
    C, H, W = inp.shape
    perm = jax.random.permutation(jax.random.key(42), C).astype(jnp.int32)
    grid_spec = pltpu.PrefetchScalarGridSpec(
        num_scalar_prefetch=1,
        grid=(1,),
        in_specs=[pl.BlockSpec(memory_space=pltpu.MemorySpace.HBM)],
        out_specs=pl.BlockSpec(memory_space=pltpu.MemorySpace.HBM),
        scratch_shapes=[pltpu.SemaphoreType.DMA((_W,))],
    )
    return pl.pallas_call(
        _dma_body,
        grid_spec=grid_spec,
        out_shape=jax.ShapeDtypeStruct((C, H, W), inp.dtype),
    )(perm, inp)


# speedup vs baseline: 12.3494x; 12.3494x over previous
import functools

import jax
import jax.numpy as jnp
from jax import lax
from jax.experimental import pallas as pl
from jax.experimental.pallas import tpu as pltpu
from jax.experimental.pallas import tpu_sc as plsc

_C, _H, _W = 768, 224, 224
_NC, _NS = 2, 16
_NW = _NC * _NS
_RPW = _C // _NW


def _sc_body(inp_hbm, idx_hbm, out_hbm, idx_v, buf, gsem, ssem):
    wid = lax.axis_index("s") * _NC + lax.axis_index("c")
    base = wid * _RPW
    pltpu.sync_copy(idx_hbm.at[pl.ds(base, _RPW)], idx_v)
    v0 = idx_v[pl.ds(0, 16)]
    v1 = idx_v[pl.ds(8, 16)]

    def gather(i):
        p = v0[i] if i < 16 else v1[i - 8]
        return pltpu.make_async_copy(
            inp_hbm.at[pl.ds(p, 1)],
            buf.at[pl.ds(i % 2, 1)],
            gsem.at[i % 2],
        )

    def scatter(i):
        return pltpu.make_async_copy(
            buf.at[pl.ds(i % 2, 1)],
            out_hbm.at[pl.ds(base + i, 1)],
            ssem.at[i % 2],
        )

    for i in range(_RPW):
        if i >= 2:
            scatter(i - 2).wait()
        gather(i).start()
        gather(i).wait()
        scatter(i).start()
    scatter(_RPW - 2).wait()
    scatter(_RPW - 1).wait()


@functools.partial(jax.jit, donate_argnums=())
def _sc_permute(inp, idx_pad):
    mesh = plsc.VectorSubcoreMesh(
        core_axis_name="c", subcore_axis_name="s", num_cores=_NC, num_subcores=_NS
    )
    return pl.kernel(
        _sc_body,
        out_type=jax.ShapeDtypeStruct((_C, _H, _W), jnp.float32),
        mesh=mesh,
        scratch_types=[
            pltpu.VMEM((_RPW,), jnp.int32),
            pltpu.VMEM((2, _H, _W), jnp.float32),
            pltpu.SemaphoreType.DMA((2,)),
            pltpu.SemaphoreType.DMA((2,)),
        ],
    )(inp, idx_pad)


def kernel(inp):
    perm = jax.random.permutation(jax.random.key(42), _C).astype(jnp.int32)
    return _sc_permute(inp, perm)


# speedup vs baseline: 16.1500x; 1.3078x over previous
import functools

import jax
import jax.numpy as jnp
from jax import lax
from jax.experimental import pallas as pl
from jax.experimental.pallas import tpu as pltpu
from jax.experimental.pallas import tpu_sc as plsc

_C = 768
_J = 224 * 224
_NC, _NS = 2, 16
_NW = _NC * _NS
_JPW = _J // _NW
_B = 32
_NCHUNK = _JPW // _B
_G = _C // 16


def _sc_body(x_hbm, idx_hbm, out_hbm, idx_v, inbuf, outbuf, gsem, ssem):
    wid = lax.axis_index("s") * _NC + lax.axis_index("c")
    base = wid * _JPW
    pltpu.sync_copy(idx_hbm, idx_v)

    def in_copy(k, b):
        return pltpu.make_async_copy(
            x_hbm.at[pl.ds(base + k * _B, _B)], inbuf.at[b], gsem.at[b]
        )

    def out_copy(k, b):
        return pltpu.make_async_copy(
            outbuf.at[b], out_hbm.at[pl.ds(base + k * _B, _B)], ssem.at[b]
        )

    cvecs = [idx_v[pl.ds(g * 16, 16)] for g in range(_G)]

    def make_row_body(b):
        def row_body(j, c2):
            jvec = jnp.full((16,), j, dtype=jnp.int32)
            for g in range(_G):
                v = plsc.load_gather(inbuf.at[b], [jvec, cvecs[g]])
                outbuf[b, j, pl.ds(g * 16, 16)] = v
            return c2

        return row_body

    row_bodies = [make_row_body(0), make_row_body(1)]

    in_copy(0, 0).start()

    def pair_body(p, carry):
        for sub in range(2):
            k = 2 * p + sub

            @pl.when(k < _NCHUNK)
            def _():
                in_copy(k, sub).wait()

                @pl.when(k + 1 < _NCHUNK)
                def _():
                    in_copy(k + 1, 1 - sub).start()

                @pl.when(k >= 2)
                def _():
                    out_copy(k - 2, sub).wait()

                lax.fori_loop(0, _B, row_bodies[sub], 0)
                out_copy(k, sub).start()

        return carry

    lax.fori_loop(0, (_NCHUNK + 1) // 2, pair_body, 0)
    out_copy(_NCHUNK - 2, (_NCHUNK - 2) % 2).wait()
    out_copy(_NCHUNK - 1, (_NCHUNK - 1) % 2).wait()


@jax.jit
def _sc_permute(x2, idx):
    mesh = plsc.VectorSubcoreMesh(
        core_axis_name="c", subcore_axis_name="s", num_cores=_NC, num_subcores=_NS
    )
    return pl.kernel(
        _sc_body,
        out_type=jax.ShapeDtypeStruct((_J, _C), jnp.float32),
        mesh=mesh,
        compiler_params=pltpu.CompilerParams(needs_layout_passes=False),
        scratch_types=[
            pltpu.VMEM((_C,), jnp.int32),
            pltpu.VMEM((2, _B, _C), jnp.float32),
            pltpu.VMEM((2, _B, _C), jnp.float32),
            pltpu.SemaphoreType.DMA((2,)),
            pltpu.SemaphoreType.DMA((2,)),
        ],
    )(x2, idx)


def kernel(inp):
    C, H, W = inp.shape
    perm = jax.random.permutation(jax.random.key(42), C).astype(jnp.int32)
    x2 = jnp.transpose(inp, (1, 2, 0)).reshape(H * W, C)
    y2 = _sc_permute(x2, perm)
    return jnp.transpose(y2.reshape(H, W, C), (2, 0, 1))


# speedup vs baseline: 39.9629x; 2.4745x over previous
import functools

import jax
import jax.numpy as jnp
from jax import lax
from jax.experimental import pallas as pl
from jax.experimental.pallas import tpu as pltpu
from jax.experimental.pallas import tpu_sc as plsc

_C = 768
_J = 224 * 224
_NC, _NS = 2, 16
_NW = _NC * _NS
_JPW = _J // _NW
_B = 32
_NCHUNK = _JPW // _B
_G = _C // 16


def _sc_body(x_hbm, idx_hbm, out_hbm, idx_v, inbuf, outbuf, gsem, ssem):
    wid = lax.axis_index("s") * _NC + lax.axis_index("c")
    base = wid * _JPW
    pltpu.sync_copy(idx_hbm, idx_v)

    def in_copy(k, b):
        return pltpu.make_async_copy(
            x_hbm.at[pl.ds(base + k * _B, _B)], inbuf.at[b], gsem.at[b]
        )

    def out_copy(k, b):
        return pltpu.make_async_copy(
            outbuf.at[b], out_hbm.at[pl.ds(base + k * _B, _B)], ssem.at[b]
        )

    cvecs = [idx_v[pl.ds(g * 16, 16)] for g in range(_G)]

    def make_row_body(b):
        def row_body(j, c2):
            jvec = jnp.full((16,), j, dtype=jnp.int32)
            for g0 in range(0, _G, 8):
                vs = [
                    plsc.load_gather(inbuf.at[b], [jvec, cvecs[g]])
                    for g in range(g0, g0 + 8)
                ]
                for g, v in zip(range(g0, g0 + 8), vs):
                    outbuf[b, j, pl.ds(g * 16, 16)] = v
            return c2

        return row_body

    row_bodies = [make_row_body(0), make_row_body(1)]

    in_copy(0, 0).start()

    def pair_body(p, carry):
        for sub in range(2):
            k = 2 * p + sub

            @pl.when(k < _NCHUNK)
            def _():
                in_copy(k, sub).wait()

                @pl.when(k + 1 < _NCHUNK)
                def _():
                    in_copy(k + 1, 1 - sub).start()

                @pl.when(k >= 2)
                def _():
                    out_copy(k - 2, sub).wait()

                lax.fori_loop(0, _B, row_bodies[sub], 0)
                out_copy(k, sub).start()

        return carry

    lax.fori_loop(0, (_NCHUNK + 1) // 2, pair_body, 0)
    out_copy(_NCHUNK - 2, (_NCHUNK - 2) % 2).wait()
    out_copy(_NCHUNK - 1, (_NCHUNK - 1) % 2).wait()


@jax.jit
def _sc_permute(x2, idx):
    mesh = plsc.VectorSubcoreMesh(
        core_axis_name="c", subcore_axis_name="s", num_cores=_NC, num_subcores=_NS
    )
    return pl.kernel(
        _sc_body,
        out_type=jax.ShapeDtypeStruct((_J, _C), jnp.float32),
        mesh=mesh,
        compiler_params=pltpu.CompilerParams(needs_layout_passes=False),
        scratch_types=[
            pltpu.VMEM((_C,), jnp.int32),
            pltpu.VMEM((2, _B, _C), jnp.float32),
            pltpu.VMEM((2, _B, _C), jnp.float32),
            pltpu.SemaphoreType.DMA((2,)),
            pltpu.SemaphoreType.DMA((2,)),
        ],
    )(x2, idx)


def kernel(inp):
    C, H, W = inp.shape
    perm = jax.random.permutation(jax.random.key(42), C).astype(jnp.int32)
    x2 = jnp.transpose(inp, (1, 2, 0)).reshape(H * W, C)
    y2 = _sc_permute(x2, perm)
    return jnp.transpose(y2.reshape(H, W, C), (2, 0, 1))


# speedup vs baseline: 40.2505x; 1.0072x over previous
import functools

import jax
import jax.numpy as jnp
from jax import lax
from jax.experimental import pallas as pl
from jax.experimental.pallas import tpu as pltpu
from jax.experimental.pallas import tpu_sc as plsc

_C = 768
_J = 224 * 224
_NC, _NS = 2, 16
_NW = _NC * _NS
_JPW = _J // _NW
_B = 32
_NCHUNK = _JPW // _B
_G = _C // 16


def _sc_body(x_hbm, idx_hbm, out_hbm, idx_v, inbuf, outbuf, gsem, ssem):
    wid = lax.axis_index("s") * _NC + lax.axis_index("c")
    base = wid * _JPW
    pltpu.sync_copy(idx_hbm, idx_v)

    def in_copy(k, b):
        return pltpu.make_async_copy(
            x_hbm.at[pl.ds(base + k * _B, _B)], inbuf.at[b], gsem.at[b]
        )

    def out_copy(k, b):
        return pltpu.make_async_copy(
            outbuf.at[b], out_hbm.at[pl.ds(base + k * _B, _B)], ssem.at[b]
        )

    cvecs = [idx_v[pl.ds(g * 16, 16)] for g in range(_G)]

    def make_row_body(b):
        def row_body(j, c2):
            jvec = jnp.full((16,), j, dtype=jnp.int32)

            def loads(g0):
                return [
                    plsc.load_gather(inbuf.at[b], [jvec, cvecs[g]])
                    for g in range(g0, g0 + 8)
                ]

            def stores(g0, vs):
                for g, v in zip(range(g0, g0 + 8), vs):
                    outbuf[b, j, pl.ds(g * 16, 16)] = v

            prev = loads(0)
            for g0 in range(8, _G, 8):
                cur = loads(g0)
                stores(g0 - 8, prev)
                prev = cur
            stores(_G - 8, prev)
            return c2

        return row_body

    row_bodies = [make_row_body(0), make_row_body(1)]

    in_copy(0, 0).start()

    def pair_body(p, carry):
        for sub in range(2):
            k = 2 * p + sub

            @pl.when(k < _NCHUNK)
            def _():
                in_copy(k, sub).wait()

                @pl.when(k + 1 < _NCHUNK)
                def _():
                    in_copy(k + 1, 1 - sub).start()

                @pl.when(k >= 2)
                def _():
                    out_copy(k - 2, sub).wait()

                lax.fori_loop(0, _B, row_bodies[sub], 0)
                out_copy(k, sub).start()

        return carry

    lax.fori_loop(0, (_NCHUNK + 1) // 2, pair_body, 0)
    out_copy(_NCHUNK - 2, (_NCHUNK - 2) % 2).wait()
    out_copy(_NCHUNK - 1, (_NCHUNK - 1) % 2).wait()


@jax.jit
def _sc_permute(x2, idx):
    mesh = plsc.VectorSubcoreMesh(
        core_axis_name="c", subcore_axis_name="s", num_cores=_NC, num_subcores=_NS
    )
    return pl.kernel(
        _sc_body,
        out_type=jax.ShapeDtypeStruct((_J, _C), jnp.float32),
        mesh=mesh,
        compiler_params=pltpu.CompilerParams(needs_layout_passes=False),
        scratch_types=[
            pltpu.VMEM((_C,), jnp.int32),
            pltpu.VMEM((2, _B, _C), jnp.float32),
            pltpu.VMEM((2, _B, _C), jnp.float32),
            pltpu.SemaphoreType.DMA((2,)),
            pltpu.SemaphoreType.DMA((2,)),
        ],
    )(x2, idx)


def kernel(inp):
    C, H, W = inp.shape
    perm = jax.random.permutation(jax.random.key(42), C).astype(jnp.int32)
    x2 = jnp.transpose(inp, (1, 2, 0)).reshape(H * W, C)
    y2 = _sc_permute(x2, perm)
    return jnp.transpose(y2.reshape(H, W, C), (2, 0, 1))
